# baseline (speedup 1.0000x reference)
import jax
import jax.numpy as jnp
from jax import lax
from jax.experimental import pallas as pl
from jax.experimental.pallas import tpu as pltpu, tpu_sc as plsc

NC, NS, L = 2, 16, 16
NW = NC * NS
B, A, C = 32, 65536, 8
KT = A // 128
CK = 16
NCHUNK = KT // CK
ROWS = CK * C


def _sc_body(lp_hbm, lt_hbm, ct_hbm, out_hbm,
             lp0, lt0, ct0, lp1, lt1, ct1, res_buf, sem0, sem1):
    b = lax.axis_index("s") * NC + lax.axis_index("c")
    r = b // 8
    i = b % 8
    bufs = ((lp0, lt0, ct0, sem0), (lp1, lt1, ct1, sem1))

    def descs(c0, bufset):
        lpb, ltb, ctb, sem = bufset
        return (
            pltpu.make_async_copy(
                lp_hbm.at[b, pl.ds(c0 * ROWS, ROWS), :], lpb, sem),
            pltpu.make_async_copy(
                lt_hbm.at[b, pl.ds(c0 * ROWS, ROWS), :], ltb, sem),
            pltpu.make_async_copy(
                ct_hbm.at[r, pl.ds(c0 * CK, CK), i, :], ctb, sem),
        )

    def start(c0, bufset):
        for d_ in descs(c0, bufset):
            d_.start()

    def wait(c0, bufset):
        for d_ in descs(c0, bufset):
            d_.wait()

    def compute(bufset, carry):
        lpb, ltb, ctb, _ = bufset

        def tile_body(kk, carry):
            accq = list(carry[0:4])
            accu = list(carry[4:8])
            cnt = carry[8]
            m = []
            for l in range(8):
                tl = ctb[kk, pl.ds(l * L, L)]
                ml = jnp.where(tl > 0, 1.0, 0.0).astype(jnp.float32)
                cnt = cnt + ml
                m.append(ml)
            for c in range(8):
                row = kk * 8 + c
                for l in range(8):
                    a = lpb[row, pl.ds(l * L, L)]
                    bb = ltb[row, pl.ds(l * L, L)]
                    d = (a - bb) * m[l]
                    absd = jnp.abs(d)
                    t = jnp.minimum(absd, 1.0)
                    j = l % 4
                    accq[j] = accq[j] + (0.5 * t) * t
                    accu[j] = accu[j] + (absd - t)
            return (*accq, *accu, cnt)

        return lax.fori_loop(0, CK, tile_body, carry)

    start(0, bufs[0])
    z = jnp.zeros((L,), jnp.float32)
    carry = (z,) * 9

    def g_body(g, carry):
        c0 = 2 * g
        wait(c0, bufs[0])
        start(c0 + 1, bufs[1])
        carry = compute(bufs[0], carry)
        wait(c0 + 1, bufs[1])
        start(jnp.minimum(c0 + 2, NCHUNK - 2), bufs[0])
        carry = compute(bufs[1], carry)
        return carry

    carry = lax.fori_loop(0, NCHUNK // 2, g_body, carry)
    wait(0, bufs[0])
    sl1 = (carry[0] + carry[1]) + (carry[2] + carry[3]) \
        + (carry[4] + carry[5]) + (carry[6] + carry[7])
    res_buf[pl.ds(0, L)] = sl1
    res_buf[pl.ds(L, L)] = carry[8]
    pltpu.sync_copy(res_buf, out_hbm.at[b])


def kernel(loc_preds, loc_targets, cls_preds, cls_targets):
    lp = loc_preds.reshape(B, KT, 128, C).transpose(0, 1, 3, 2).reshape(B, KT * C, 128)
    lt = loc_targets.reshape(B, KT, 128, C).transpose(0, 1, 3, 2).reshape(B, KT * C, 128)
    ct = cls_targets.astype(jnp.int32).reshape(B // 8, 8, KT, 128).transpose(0, 2, 1, 3)
    mesh = plsc.VectorSubcoreMesh(
        core_axis_name="c", subcore_axis_name="s",
        num_cores=NC, num_subcores=NS)
    out = pl.kernel(
        _sc_body,
        out_type=jax.ShapeDtypeStruct((NW, 2 * L), jnp.float32),
        mesh=mesh,
        scratch_types=[
            pltpu.VMEM((ROWS, 128), jnp.float32),
            pltpu.VMEM((ROWS, 128), jnp.float32),
            pltpu.VMEM((CK, 128), jnp.int32),
            pltpu.VMEM((ROWS, 128), jnp.float32),
            pltpu.VMEM((ROWS, 128), jnp.float32),
            pltpu.VMEM((CK, 128), jnp.int32),
            pltpu.VMEM((2 * L,), jnp.float32),
            pltpu.SemaphoreType.DMA,
            pltpu.SemaphoreType.DMA,
        ],
    )(lp, lt, ct)
    sl1_sum = jnp.sum(out[:, :L])
    n = jnp.sum(out[:, L:])
    return 0.2 * (sl1_sum / n)
